# Initial kernel scaffold; baseline (speedup 1.0000x reference)
import jax
import jax.numpy as jnp
from jax.experimental import pallas as pl


def kernel(x, edge_index, edge_weight, batch_vector, W_enc, b_enc, W_layers, b_layers, W_dec, b_dec):
    raise NotImplementedError("write your pallas kernel here")


# speedup vs baseline: 4.0911x; 4.0911x over previous
import functools

import jax
import jax.numpy as jnp
from jax import lax
from jax.experimental import pallas as pl
from jax.experimental.pallas import tpu as pltpu
from jax.experimental.pallas import tpu_sc as plsc

N = 10000
E = 320000
H = 128
L = 4

NC = 2
NS = 16
NW = NC * NS
EPW = E // NW
C = 80
NCHUNK = EPW // C
RPB = 624
REM = N - NS * RPB

_mesh = plsc.VectorSubcoreMesh(core_axis_name="c", subcore_axis_name="s")

_GDN = lax.GatherDimensionNumbers(
    offset_dims=(), collapsed_slice_dims=(0,), start_index_map=(0,))


def _splat_lane(v16, lane):
    idx = jnp.full((16, 1), lane, jnp.int32)
    return lax.gather(v16, idx, _GDN, (1,),
                      mode=lax.GatherScatterMode.PROMISE_IN_BOUNDS)


@functools.partial(
    pl.kernel,
    out_type=jax.ShapeDtypeStruct((NC, N, H), jnp.float32),
    mesh=_mesh,
    scratch_types=[
        pltpu.VMEM((C,), jnp.int32),
        pltpu.VMEM((C,), jnp.int32),
        pltpu.VMEM((C,), jnp.float32),
        pltpu.VMEM((C, H), jnp.float32),
        pltpu.VMEM_SHARED((N, H), jnp.float32),
        pltpu.SemaphoreType.DMA,
    ],
)
def _sc_message(h_hbm, src_hbm, dst_hbm, w_hbm, out_hbm,
                src_v, dst_v, w_v, rows_v, agg_sh, sem):
    cid = lax.axis_index("c")
    sid = lax.axis_index("s")
    wid = sid * NC + cid

    zeros16 = jnp.zeros((16,), jnp.float32)

    def _zero_row(i, _):
        for j in range(H // 16):
            rows_v[i, pl.ds(j * 16, 16)] = zeros16
        return 0

    lax.fori_loop(0, C, _zero_row, 0)
    for t in range(RPB // C):
        pltpu.sync_copy(rows_v, agg_sh.at[pl.ds(sid * RPB + t * C, C)])
    tail = RPB - (RPB // C) * C
    pltpu.sync_copy(rows_v.at[pl.ds(0, tail)],
                    agg_sh.at[pl.ds(sid * RPB + RPB - tail, tail)])

    @pl.when(sid == 0)
    def _zero_rem():
        pltpu.sync_copy(rows_v.at[pl.ds(0, REM)],
                        agg_sh.at[pl.ds(NS * RPB, REM)])

    plsc.subcore_barrier()

    def _chunk(k, _):
        base = wid * EPW + k * C
        pltpu.sync_copy(src_hbm.at[pl.ds(base, C)], src_v)
        pltpu.sync_copy(dst_hbm.at[pl.ds(base, C)], dst_v)
        pltpu.sync_copy(w_hbm.at[pl.ds(base, C)], w_v)
        pltpu.async_copy(h_hbm.at[src_v], rows_v, sem).wait()

        def _scale(g, _):
            w16 = w_v[pl.ds(g * 16, 16)]
            for l in range(16):
                wl = _splat_lane(w16, l)
                e = g * 16 + l
                for j in range(H // 16):
                    rows_v[e, pl.ds(j * 16, 16)] = (
                        rows_v[e, pl.ds(j * 16, 16)] * wl)
            return 0

        lax.fori_loop(0, C // 16, _scale, 0)
        pltpu.sync_copy(rows_v, agg_sh.at[dst_v], add=True)
        return 0

    lax.fori_loop(0, NCHUNK, _chunk, 0)
    plsc.subcore_barrier()

    pltpu.sync_copy(agg_sh.at[pl.ds(sid * RPB, RPB)],
                    out_hbm.at[cid, pl.ds(sid * RPB, RPB)])

    @pl.when(sid == 0)
    def _copy_rem():
        pltpu.sync_copy(agg_sh.at[pl.ds(NS * RPB, REM)],
                        out_hbm.at[cid, pl.ds(NS * RPB, REM)])


BR = 2000


def _enc_body(x_ref, w_ref, b_ref, o_ref):
    o_ref[...] = jnp.dot(x_ref[...], w_ref[...],
                         preferred_element_type=jnp.float32) + b_ref[...]


_tc_encoder = pl.pallas_call(
    _enc_body,
    grid=(N // BR,),
    in_specs=[
        pl.BlockSpec((BR, H), lambda i: (i, 0)),
        pl.BlockSpec((H, H), lambda i: (0, 0)),
        pl.BlockSpec((1, H), lambda i: (0, 0)),
    ],
    out_specs=pl.BlockSpec((BR, H), lambda i: (i, 0)),
    out_shape=jax.ShapeDtypeStruct((N, H), jnp.float32),
)


def _layer_body(p_ref, w_ref, b_ref, o_ref):
    agg = p_ref[0] + p_ref[1]
    o_ref[...] = jnp.maximum(
        jnp.dot(agg, w_ref[...], preferred_element_type=jnp.float32)
        + b_ref[...], 0.0)


_tc_layer = pl.pallas_call(
    _layer_body,
    grid=(N // BR,),
    in_specs=[
        pl.BlockSpec((NC, BR, H), lambda i: (0, i, 0)),
        pl.BlockSpec((H, H), lambda i: (0, 0)),
        pl.BlockSpec((1, H), lambda i: (0, 0)),
    ],
    out_specs=pl.BlockSpec((BR, H), lambda i: (i, 0)),
    out_shape=jax.ShapeDtypeStruct((N, H), jnp.float32),
)


def _last_body(p_ref, w_ref, b_ref, wd_ref, bd_ref, o_ref):
    agg = p_ref[0] + p_ref[1]
    h = jnp.maximum(
        jnp.dot(agg, w_ref[...], preferred_element_type=jnp.float32)
        + b_ref[...], 0.0)
    o_ref[...] = jnp.dot(h, wd_ref[...],
                         preferred_element_type=jnp.float32) + bd_ref[...]


_tc_last = pl.pallas_call(
    _last_body,
    grid=(N // BR,),
    in_specs=[
        pl.BlockSpec((NC, BR, H), lambda i: (0, i, 0)),
        pl.BlockSpec((H, H), lambda i: (0, 0)),
        pl.BlockSpec((1, H), lambda i: (0, 0)),
        pl.BlockSpec((H, H), lambda i: (0, 0)),
        pl.BlockSpec((1, H), lambda i: (0, 0)),
    ],
    out_specs=pl.BlockSpec((BR, H), lambda i: (i, 0)),
    out_shape=jax.ShapeDtypeStruct((N, H), jnp.float32),
)


def kernel(x, edge_index, edge_weight, batch_vector,
           W_enc, b_enc, W_layers, b_layers, W_dec, b_dec):
    xf = x.reshape(N, -1)
    src = edge_index[0]
    dst = edge_index[1]

    h = _tc_encoder(xf, W_enc, b_enc.reshape(1, H))

    wd = jnp.zeros((H, H), jnp.float32).at[:, :1].set(W_dec)
    bd = jnp.zeros((1, H), jnp.float32).at[:, :1].set(b_dec.reshape(1, 1))

    for i in range(L):
        partials = _sc_message(h, src, dst, edge_weight)
        if i < L - 1:
            h = _tc_layer(partials, W_layers[i], b_layers[i].reshape(1, H))
        else:
            out_full = _tc_last(partials, W_layers[i],
                                b_layers[i].reshape(1, H), wd, bd)
    return out_full[:, :1][:, :, None]
